# baseline (speedup 1.0000x reference)
import jax
import jax.numpy as jnp
from jax.experimental import pallas as pl
from jax.experimental.pallas import tpu as pltpu

_N = 4096
_D = 64
_P = 2
_BLK = 512
_LAM = 0.7
_EPS = 0.1
_NORM_EPS = 1e-12


def _graph_block_kernel(emb_ref, w_ref, adj_ref, out_ref, normed_ref):
    i = pl.program_id(0)

    @pl.when(i == 0)
    def _():
        weighted = emb_ref[...][None, :, :] * w_ref[...][:, None, :]
        norm = jnp.sqrt(jnp.sum(weighted * weighted, axis=-1, keepdims=True))
        normed_ref[...] = weighted / jnp.maximum(norm, _NORM_EPS)

    dn = (((1,), (1,)), ((), ()))
    sim = jnp.zeros((_BLK, _N), dtype=jnp.float32)
    for p in range(_P):
        rows = normed_ref[p, pl.ds(i * _BLK, _BLK), :]
        cols = normed_ref[p, :, :]
        sim += jax.lax.dot_general(rows, cols, dn,
                                   preferred_element_type=jnp.float32)
    sim *= 1.0 / _P
    masked = jnp.where(sim > _EPS, sim, 0.0)
    out_ref[...] = _LAM * adj_ref[...] + (1.0 - _LAM) * masked


def _build_graph(adj, emb, W, interpret=False):
    nb = _N // _BLK
    return pl.pallas_call(
        _graph_block_kernel,
        grid=(nb,),
        in_specs=[
            pl.BlockSpec((_N, _D), lambda i: (0, 0)),
            pl.BlockSpec((_P, _D), lambda i: (0, 0)),
            pl.BlockSpec((_BLK, _N), lambda i: (i, 0)),
        ],
        out_specs=pl.BlockSpec((_BLK, _N), lambda i: (i, 0)),
        out_shape=jax.ShapeDtypeStruct((_N, _N), jnp.float32),
        scratch_shapes=[pltpu.VMEM((_P, _N, _D), jnp.float32)],
        interpret=interpret,
    )(emb, W, adj)


def kernel(u2u_adj, i2i_adj, multi_u2i_adj, user_embedding, item_embedding,
           W_user, W_item):
    new_u2u = _build_graph(u2u_adj, user_embedding, W_user)
    new_i2i = _build_graph(i2i_adj, item_embedding, W_item)
    return (new_u2u, new_i2i, multi_u2i_adj)
